# initial kernel scaffold (unmeasured)
import functools

import jax
import jax.numpy as jnp
from jax import lax
from jax.experimental import pallas as pl
from jax.experimental.pallas import tpu as pltpu

N_DEV = 32
B_LOC = 2
SQ = 128
SKV = 128
H_LOC = 4
DH = 64
D_MODEL = 512
CH = H_LOC * DH


def _body(x_ref, wq_ref, wo_ref, k_ref, v_ref, out_ref,
          comm_wq, comm_wo, ctx_sc, acc_sc,
          send_q, recv_q, send_o, recv_o):
    my = lax.axis_index("i")
    right = lax.rem(my + 1, N_DEV)
    left = lax.rem(my + N_DEV - 1, N_DEV)

    barrier_sem = pltpu.get_barrier_semaphore()
    for nbr in (left, right):
        pl.semaphore_signal(
            barrier_sem, inc=1,
            device_id=(nbr,), device_id_type=pl.DeviceIdType.MESH,
        )
    pl.semaphore_wait(barrier_sem, 2)

    comm_wq[0, :, :] = wq_ref[:, :]
    comm_wo[0, :, :] = wo_ref[:, :]
    acc_sc[:, :] = jnp.zeros((B_LOC * SQ, D_MODEL), jnp.float32)

    x_flat = x_ref[:, :, :].reshape(B_LOC * SQ, D_MODEL)

    for h in range(N_DEV):
        if h < N_DEV - 1:
            rq = pltpu.make_async_remote_copy(
                src_ref=comm_wq.at[h],
                dst_ref=comm_wq.at[h + 1],
                send_sem=send_q.at[h],
                recv_sem=recv_q.at[h],
                device_id=(right,),
                device_id_type=pl.DeviceIdType.MESH,
            )
            ro = pltpu.make_async_remote_copy(
                src_ref=comm_wo.at[h],
                dst_ref=comm_wo.at[h + 1],
                send_sem=send_o.at[h],
                recv_sem=recv_o.at[h],
                device_id=(right,),
                device_id_type=pl.DeviceIdType.MESH,
            )
            rq.start()
            ro.start()

        origin = lax.rem(my + (N_DEV - h), N_DEV)
        wq_c = comm_wq[h]
        wo_c = comm_wo[h]
        q_c = jnp.dot(x_flat, wq_c, preferred_element_type=jnp.float32)

        for b in range(B_LOC):
            for hh in range(H_LOC):
                g = origin * H_LOC + hh
                q = q_c[b * SQ:(b + 1) * SQ, hh * DH:(hh + 1) * DH]
                k = k_ref[g, b]
                v = v_ref[g, b]
                s = lax.dot_general(
                    q, k, (((1,), (1,)), ((), ())),
                    preferred_element_type=jnp.float32,
                ) * 0.125
                m = jnp.max(s, axis=1, keepdims=True)
                w = jnp.exp(s - m)
                w = w / jnp.sum(w, axis=1, keepdims=True)
                ctx = jnp.dot(w, v, preferred_element_type=jnp.float32)
                ctx_sc[b * SQ:(b + 1) * SQ, hh * DH:(hh + 1) * DH] = ctx

        acc_sc[:, :] = acc_sc[:, :] + jnp.dot(
            ctx_sc[:, :], wo_c, preferred_element_type=jnp.float32
        )

        if h < N_DEV - 1:
            rq.wait_send()
            ro.wait_send()
            rq.wait_recv()
            ro.wait_recv()

    out_ref[:, :, :] = acc_sc[:, :].reshape(B_LOC, SQ, D_MODEL)


def kernel(x, Wq, K_ext, V_ext, Wo):
    my = lax.axis_index("i")
    k_loc = lax.dynamic_slice_in_dim(K_ext, my * B_LOC, B_LOC, axis=0)
    v_loc = lax.dynamic_slice_in_dim(V_ext, my * B_LOC, B_LOC, axis=0)
    k_t = jnp.transpose(k_loc, (2, 0, 1, 3))
    v_t = jnp.transpose(v_loc, (2, 0, 1, 3))

    return pl.pallas_call(
        _body,
        out_shape=jax.ShapeDtypeStruct((B_LOC, SQ, D_MODEL), jnp.float32),
        in_specs=[
            pl.BlockSpec(memory_space=pltpu.VMEM),
            pl.BlockSpec(memory_space=pltpu.VMEM),
            pl.BlockSpec(memory_space=pltpu.VMEM),
            pl.BlockSpec(memory_space=pltpu.VMEM),
            pl.BlockSpec(memory_space=pltpu.VMEM),
        ],
        out_specs=pl.BlockSpec(memory_space=pltpu.VMEM),
        scratch_shapes=[
            pltpu.VMEM((N_DEV, D_MODEL, CH), jnp.float32),
            pltpu.VMEM((N_DEV, CH, D_MODEL), jnp.float32),
            pltpu.VMEM((B_LOC * SQ, CH), jnp.float32),
            pltpu.VMEM((B_LOC * SQ, D_MODEL), jnp.float32),
            pltpu.SemaphoreType.DMA((N_DEV - 1,)),
            pltpu.SemaphoreType.DMA((N_DEV - 1,)),
            pltpu.SemaphoreType.DMA((N_DEV - 1,)),
            pltpu.SemaphoreType.DMA((N_DEV - 1,)),
        ],
        compiler_params=pltpu.CompilerParams(collective_id=0),
    )(x, Wq, Wo, k_t, v_t)


# baseline (device time: 453123 ns/iter reference)
import functools

import jax
import jax.numpy as jnp
from jax import lax
from jax.experimental import pallas as pl
from jax.experimental.pallas import tpu as pltpu

N_DEV = 32
B_LOC = 2
SQ = 128
SKV = 128
H_LOC = 4
DH = 64
D_MODEL = 512
CH = H_LOC * DH


def _body(x_ref, wq_ref, wo_ref, k_ref, v_ref, out_ref,
          comm_wq, comm_wo, ctx_sc, acc_sc, k_sc, v_sc,
          send_q, recv_q, send_o, recv_o, kv_sems):
    my = lax.axis_index("i")
    right = lax.rem(my + 1, N_DEV)
    left = lax.rem(my + N_DEV - 1, N_DEV)

    barrier_sem = pltpu.get_barrier_semaphore()
    for nbr in (left, right):
        pl.semaphore_signal(
            barrier_sem, inc=1,
            device_id=(nbr,), device_id_type=pl.DeviceIdType.MESH,
        )
    pl.semaphore_wait(barrier_sem, 2)

    comm_wq[0, :, :] = wq_ref[:, :]
    comm_wo[0, :, :] = wo_ref[:, :]
    acc_sc[:, :] = jnp.zeros((B_LOC * SQ, D_MODEL), jnp.float32)

    x_flat = x_ref[:, :, :].reshape(B_LOC * SQ, D_MODEL)

    for h in range(N_DEV):
        if h < N_DEV - 1:
            rq = pltpu.make_async_remote_copy(
                src_ref=comm_wq.at[h],
                dst_ref=comm_wq.at[h + 1],
                send_sem=send_q.at[h],
                recv_sem=recv_q.at[h],
                device_id=(right,),
                device_id_type=pl.DeviceIdType.MESH,
            )
            ro = pltpu.make_async_remote_copy(
                src_ref=comm_wo.at[h],
                dst_ref=comm_wo.at[h + 1],
                send_sem=send_o.at[h],
                recv_sem=recv_o.at[h],
                device_id=(right,),
                device_id_type=pl.DeviceIdType.MESH,
            )
            rq.start()
            ro.start()

        origin = lax.rem(my + (N_DEV - h), N_DEV)
        kdma = pltpu.make_async_copy(
            k_ref.at[pl.ds(origin * H_LOC, H_LOC)], k_sc, kv_sems.at[0]
        )
        vdma = pltpu.make_async_copy(
            v_ref.at[pl.ds(origin * H_LOC, H_LOC)], v_sc, kv_sems.at[1]
        )
        kdma.start()
        vdma.start()
        wq_c = comm_wq[h]
        wo_c = comm_wo[h]
        q_c = jnp.dot(x_flat, wq_c, preferred_element_type=jnp.float32)
        kdma.wait()
        vdma.wait()

        for b in range(B_LOC):
            for hh in range(H_LOC):
                q = q_c[b * SQ:(b + 1) * SQ, hh * DH:(hh + 1) * DH]
                k = k_sc[hh, b]
                v = v_sc[hh, b]
                s = lax.dot_general(
                    q, k, (((1,), (1,)), ((), ())),
                    preferred_element_type=jnp.float32,
                ) * 0.125
                m = jnp.max(s, axis=1, keepdims=True)
                w = jnp.exp(s - m)
                w = w / jnp.sum(w, axis=1, keepdims=True)
                ctx = jnp.dot(w, v, preferred_element_type=jnp.float32)
                ctx_sc[b * SQ:(b + 1) * SQ, hh * DH:(hh + 1) * DH] = ctx

        acc_sc[:, :] = acc_sc[:, :] + jnp.dot(
            ctx_sc[:, :], wo_c, preferred_element_type=jnp.float32
        )

        if h < N_DEV - 1:
            rq.wait_send()
            ro.wait_send()
            rq.wait_recv()
            ro.wait_recv()

    out_ref[:, :, :] = acc_sc[:, :].reshape(B_LOC, SQ, D_MODEL)


def kernel(x, Wq, K_ext, V_ext, Wo):
    my = lax.axis_index("i")
    k_loc = lax.dynamic_slice_in_dim(K_ext, my * B_LOC, B_LOC, axis=0)
    v_loc = lax.dynamic_slice_in_dim(V_ext, my * B_LOC, B_LOC, axis=0)
    k_t = jnp.transpose(k_loc, (2, 0, 1, 3))
    v_t = jnp.transpose(v_loc, (2, 0, 1, 3))

    return pl.pallas_call(
        _body,
        out_shape=jax.ShapeDtypeStruct((B_LOC, SQ, D_MODEL), jnp.float32),
        in_specs=[
            pl.BlockSpec(memory_space=pltpu.VMEM),
            pl.BlockSpec(memory_space=pltpu.VMEM),
            pl.BlockSpec(memory_space=pltpu.VMEM),
            pl.BlockSpec(memory_space=pl.ANY),
            pl.BlockSpec(memory_space=pl.ANY),
        ],
        out_specs=pl.BlockSpec(memory_space=pltpu.VMEM),
        scratch_shapes=[
            pltpu.VMEM((N_DEV, D_MODEL, CH), jnp.float32),
            pltpu.VMEM((N_DEV, CH, D_MODEL), jnp.float32),
            pltpu.VMEM((B_LOC * SQ, CH), jnp.float32),
            pltpu.VMEM((B_LOC * SQ, D_MODEL), jnp.float32),
            pltpu.VMEM((H_LOC, B_LOC, SKV, DH), jnp.float32),
            pltpu.VMEM((H_LOC, B_LOC, SKV, DH), jnp.float32),
            pltpu.SemaphoreType.DMA((N_DEV - 1,)),
            pltpu.SemaphoreType.DMA((N_DEV - 1,)),
            pltpu.SemaphoreType.DMA((N_DEV - 1,)),
            pltpu.SemaphoreType.DMA((N_DEV - 1,)),
            pltpu.SemaphoreType.DMA((2,)),
        ],
        compiler_params=pltpu.CompilerParams(
            collective_id=0,
            vmem_limit_bytes=100 * 1024 * 1024,
        ),
    )(x, Wq, Wo, k_t, v_t)


def _preflight():
    try:
        if len(jax.devices()) < N_DEV or jax.devices()[0].platform == "cpu":
            return
        import distributed_mesh_v7x as dm
        from jax.experimental.shard_map import shard_map
        from jax.sharding import NamedSharding, PartitionSpec as P

        mesh = dm.get_mesh("i", world_size=N_DEV)
        in_p = (P("i", None, None), P(None, "i"), P(None, None, None, None),
                P(None, None, None, None), P("i", None))
        shapes = ((64, 128, 512), (512, 8192), (64, 128, 128, 64),
                  (64, 128, 128, 64), (8192, 512))
        args = [jax.ShapeDtypeStruct(s, jnp.float32,
                                     sharding=NamedSharding(mesh, p))
                for s, p in zip(shapes, in_p)]
        wrapped = jax.jit(shard_map(
            kernel, mesh=mesh, in_specs=in_p,
            out_specs=P("i", None, None), check_rep=False))
        wrapped.lower(*args).compile()
    except Exception:
        pass
    try:
        for a in jax.live_arrays():
            a.block_until_ready()
    except Exception:
        pass


_preflight()


# device time: 253836 ns/iter; 1.7851x vs baseline; 1.7851x over previous
import jax
import jax.numpy as jnp
from jax import lax
from jax.experimental import pallas as pl
from jax.experimental.pallas import tpu as pltpu

N_DEV = 32
B_LOC = 2
SQ = 128
SKV = 128
H_LOC = 4
DH = 64
D_MODEL = 512
CH = H_LOC * DH

N_R = 16
N_L = 15


def _attn_chunk(x16, q_wq, q_wo, origin, k_ref, v_ref, k_sc, v_sc,
                kv_sems, ctx_sc, acc_sc):
    kdma = pltpu.make_async_copy(
        k_ref.at[pl.ds(origin * H_LOC, H_LOC)], k_sc, kv_sems.at[0])
    vdma = pltpu.make_async_copy(
        v_ref.at[pl.ds(origin * H_LOC, H_LOC)], v_sc, kv_sems.at[1])
    kdma.start()
    vdma.start()
    q_c = jnp.dot(x16, q_wq, preferred_element_type=jnp.float32)
    kdma.wait()
    vdma.wait()
    for b in range(B_LOC):
        for hh in range(H_LOC):
            q = q_c[b * SQ:(b + 1) * SQ, hh * DH:(hh + 1) * DH]
            k = k_sc[hh, b]
            v = v_sc[hh, b]
            s = lax.dot_general(
                q, k, (((1,), (1,)), ((), ())),
                preferred_element_type=jnp.float32,
            ) * 0.125
            m = jnp.max(s, axis=1, keepdims=True)
            w = jnp.exp(s - m)
            w = w / jnp.sum(w, axis=1, keepdims=True)
            ctx = jnp.dot(w, v, preferred_element_type=jnp.float32)
            ctx_sc[b * SQ:(b + 1) * SQ, hh * DH:(hh + 1) * DH] = ctx
    acc_sc[:, :] = acc_sc[:, :] + jnp.dot(
        ctx_sc[:, :].astype(jnp.bfloat16), q_wo,
        preferred_element_type=jnp.float32)


def _body(x_ref, wq_ref, wo_ref, k_ref, v_ref, out_ref,
          comm_rq, comm_ro, comm_lq, comm_lo, ctx_sc, acc_sc,
          k_r, v_r, k_l, v_l,
          s_rq, r_rq, s_ro, r_ro, s_lq, r_lq, s_lo, r_lo, kv_sems):
    my = lax.axis_index("i")
    right = lax.rem(my + 1, N_DEV)
    left = lax.rem(my + N_DEV - 1, N_DEV)

    barrier_sem = pltpu.get_barrier_semaphore()
    for nbr in (left, right):
        pl.semaphore_signal(
            barrier_sem, inc=1,
            device_id=(nbr,), device_id_type=pl.DeviceIdType.MESH,
        )
    pl.semaphore_wait(barrier_sem, 2)

    comm_rq[0, :, :] = wq_ref[:, :]
    comm_ro[0, :, :] = wo_ref[:, :]
    comm_lq[0, :, :] = wq_ref[:, :]
    comm_lo[0, :, :] = wo_ref[:, :]
    acc_sc[:, :] = jnp.zeros((B_LOC * SQ, D_MODEL), jnp.float32)

    x16 = x_ref[:, :, :].reshape(B_LOC * SQ, D_MODEL).astype(jnp.bfloat16)

    for h in range(N_R + 1):
        rdmas = []
        if h < N_R:
            for (src, sem_s, sem_r) in ((comm_rq, s_rq, r_rq),
                                        (comm_ro, s_ro, r_ro)):
                r = pltpu.make_async_remote_copy(
                    src_ref=src.at[h], dst_ref=src.at[h + 1],
                    send_sem=sem_s.at[h], recv_sem=sem_r.at[h],
                    device_id=(right,), device_id_type=pl.DeviceIdType.MESH)
                r.start()
                rdmas.append(r)
        if h < N_L:
            for (src, sem_s, sem_r) in ((comm_lq, s_lq, r_lq),
                                        (comm_lo, s_lo, r_lo)):
                r = pltpu.make_async_remote_copy(
                    src_ref=src.at[h], dst_ref=src.at[h + 1],
                    send_sem=sem_s.at[h], recv_sem=sem_r.at[h],
                    device_id=(left,), device_id_type=pl.DeviceIdType.MESH)
                r.start()
                rdmas.append(r)

        origin_r = lax.rem(my + (N_DEV - h), N_DEV)
        _attn_chunk(x16, comm_rq[h], comm_ro[h], origin_r,
                    k_ref, v_ref, k_r, v_r, kv_sems, ctx_sc, acc_sc)
        if 0 < h < N_L + 1:
            origin_l = lax.rem(my + h, N_DEV)
            _attn_chunk(x16, comm_lq[h], comm_lo[h], origin_l,
                        k_ref, v_ref, k_l, v_l, kv_sems, ctx_sc, acc_sc)

        for r in rdmas:
            r.wait_send()
        for r in rdmas:
            r.wait_recv()

    out_ref[:, :, :] = acc_sc[:, :].reshape(B_LOC, SQ, D_MODEL)


def kernel(x, Wq, K_ext, V_ext, Wo):
    my = lax.axis_index("i")
    k_loc = lax.dynamic_slice_in_dim(K_ext, my * B_LOC, B_LOC, axis=0)
    v_loc = lax.dynamic_slice_in_dim(V_ext, my * B_LOC, B_LOC, axis=0)
    k_t = jnp.transpose(k_loc, (2, 0, 1, 3))
    v_t = jnp.transpose(v_loc, (2, 0, 1, 3))

    bf = jnp.bfloat16
    return pl.pallas_call(
        _body,
        out_shape=jax.ShapeDtypeStruct((B_LOC, SQ, D_MODEL), jnp.float32),
        in_specs=[
            pl.BlockSpec(memory_space=pltpu.VMEM),
            pl.BlockSpec(memory_space=pltpu.VMEM),
            pl.BlockSpec(memory_space=pltpu.VMEM),
            pl.BlockSpec(memory_space=pl.ANY),
            pl.BlockSpec(memory_space=pl.ANY),
        ],
        out_specs=pl.BlockSpec(memory_space=pltpu.VMEM),
        scratch_shapes=[
            pltpu.VMEM((N_R + 1, D_MODEL, CH), bf),
            pltpu.VMEM((N_R + 1, CH, D_MODEL), bf),
            pltpu.VMEM((N_L + 1, D_MODEL, CH), bf),
            pltpu.VMEM((N_L + 1, CH, D_MODEL), bf),
            pltpu.VMEM((B_LOC * SQ, CH), jnp.float32),
            pltpu.VMEM((B_LOC * SQ, D_MODEL), jnp.float32),
            pltpu.VMEM((H_LOC, B_LOC, SKV, DH), jnp.float32),
            pltpu.VMEM((H_LOC, B_LOC, SKV, DH), jnp.float32),
            pltpu.VMEM((H_LOC, B_LOC, SKV, DH), jnp.float32),
            pltpu.VMEM((H_LOC, B_LOC, SKV, DH), jnp.float32),
            pltpu.SemaphoreType.DMA((N_R,)),
            pltpu.SemaphoreType.DMA((N_R,)),
            pltpu.SemaphoreType.DMA((N_R,)),
            pltpu.SemaphoreType.DMA((N_R,)),
            pltpu.SemaphoreType.DMA((N_L,)),
            pltpu.SemaphoreType.DMA((N_L,)),
            pltpu.SemaphoreType.DMA((N_L,)),
            pltpu.SemaphoreType.DMA((N_L,)),
            pltpu.SemaphoreType.DMA((2,)),
        ],
        compiler_params=pltpu.CompilerParams(
            collective_id=0,
            vmem_limit_bytes=100 * 1024 * 1024,
        ),
    )(x, Wq.astype(bf), Wo.astype(bf), k_t, v_t)


def _preflight():
    try:
        if len(jax.devices()) < N_DEV or jax.devices()[0].platform == "cpu":
            return
        import distributed_mesh_v7x as dm
        from jax.experimental.shard_map import shard_map
        from jax.sharding import NamedSharding, PartitionSpec as P

        mesh = dm.get_mesh("i", world_size=N_DEV)
        in_p = (P("i", None, None), P(None, "i"), P(None, None, None, None),
                P(None, None, None, None), P("i", None))
        shapes = ((64, 128, 512), (512, 8192), (64, 128, 128, 64),
                  (64, 128, 128, 64), (8192, 512))
        args = [jax.ShapeDtypeStruct(s, jnp.float32,
                                     sharding=NamedSharding(mesh, p))
                for s, p in zip(shapes, in_p)]
        wrapped = jax.jit(shard_map(
            kernel, mesh=mesh, in_specs=in_p,
            out_specs=P("i", None, None), check_rep=False))
        wrapped.lower(*args).compile()
    except Exception:
        pass
    try:
        for a in jax.live_arrays():
            a.block_until_ready()
    except Exception:
        pass


_preflight()


# device time: 252593 ns/iter; 1.7939x vs baseline; 1.0049x over previous
import jax
import jax.numpy as jnp
from jax import lax
from jax.experimental import pallas as pl
from jax.experimental.pallas import tpu as pltpu

N_DEV = 32
B_LOC = 2
SQ = 128
SKV = 128
H_LOC = 4
DH = 64
D_MODEL = 512
CH = H_LOC * DH

N_R = 16
N_L = 15


def _attn_chunk(x16, q_wq, q_wo, origin, k_ref, v_ref, k_sc, v_sc,
                kv_sems, acc_sc):
    kdma = pltpu.make_async_copy(
        k_ref.at[:, pl.ds(origin * H_LOC, H_LOC)], k_sc, kv_sems.at[0])
    vdma = pltpu.make_async_copy(
        v_ref.at[:, pl.ds(origin * H_LOC, H_LOC)], v_sc, kv_sems.at[1])
    kdma.start()
    vdma.start()
    q_c = jnp.dot(x16, q_wq, preferred_element_type=jnp.float32)
    q3 = q_c.reshape(B_LOC, SQ, H_LOC, DH).transpose(0, 2, 1, 3).reshape(
        B_LOC * H_LOC, SQ, DH)
    kdma.wait()
    vdma.wait()
    k3 = k_sc[:, :, :, :].reshape(B_LOC * H_LOC, SKV, DH)
    v3 = v_sc[:, :, :, :].reshape(B_LOC * H_LOC, SKV, DH)
    s = lax.dot_general(
        q3, k3, (((2,), (2,)), ((0,), (0,))),
        preferred_element_type=jnp.float32,
    ) * 0.125
    m = jnp.max(s, axis=2, keepdims=True)
    w = jnp.exp(s - m)
    w = w / jnp.sum(w, axis=2, keepdims=True)
    ctx3 = lax.dot_general(
        w, v3, (((2,), (1,)), ((0,), (0,))),
        preferred_element_type=jnp.float32,
    )
    ctx = ctx3.reshape(B_LOC, H_LOC, SQ, DH).transpose(0, 2, 1, 3).reshape(
        B_LOC * SQ, CH)
    acc_sc[:, :] = acc_sc[:, :] + jnp.dot(
        ctx.astype(jnp.bfloat16), q_wo,
        preferred_element_type=jnp.float32)


def _body(x_ref, wq_ref, wo_ref, k_ref, v_ref, out_ref,
          comm_rq, comm_ro, comm_lq, comm_lo, acc_sc,
          k_r, v_r, k_l, v_l,
          s_rq, r_rq, s_ro, r_ro, s_lq, r_lq, s_lo, r_lo, kv_sems):
    my = lax.axis_index("i")
    right = lax.rem(my + 1, N_DEV)
    left = lax.rem(my + N_DEV - 1, N_DEV)

    barrier_sem = pltpu.get_barrier_semaphore()
    for nbr in (left, right):
        pl.semaphore_signal(
            barrier_sem, inc=1,
            device_id=(nbr,), device_id_type=pl.DeviceIdType.MESH,
        )
    pl.semaphore_wait(barrier_sem, 2)

    comm_rq[0, :, :] = wq_ref[:, :]
    comm_ro[0, :, :] = wo_ref[:, :]
    comm_lq[0, :, :] = wq_ref[:, :]
    comm_lo[0, :, :] = wo_ref[:, :]
    acc_sc[:, :] = jnp.zeros((B_LOC * SQ, D_MODEL), jnp.float32)

    x16 = x_ref[:, :, :].reshape(B_LOC * SQ, D_MODEL).astype(jnp.bfloat16)

    for h in range(N_R + 1):
        rdmas = []
        if h < N_R:
            for (src, sem_s, sem_r) in ((comm_rq, s_rq, r_rq),
                                        (comm_ro, s_ro, r_ro)):
                r = pltpu.make_async_remote_copy(
                    src_ref=src.at[h], dst_ref=src.at[h + 1],
                    send_sem=sem_s.at[h], recv_sem=sem_r.at[h],
                    device_id=(right,), device_id_type=pl.DeviceIdType.MESH)
                r.start()
                rdmas.append(r)
        if h < N_L:
            for (src, sem_s, sem_r) in ((comm_lq, s_lq, r_lq),
                                        (comm_lo, s_lo, r_lo)):
                r = pltpu.make_async_remote_copy(
                    src_ref=src.at[h], dst_ref=src.at[h + 1],
                    send_sem=sem_s.at[h], recv_sem=sem_r.at[h],
                    device_id=(left,), device_id_type=pl.DeviceIdType.MESH)
                r.start()
                rdmas.append(r)

        origin_r = lax.rem(my + (N_DEV - h), N_DEV)
        _attn_chunk(x16, comm_rq[h], comm_ro[h], origin_r,
                    k_ref, v_ref, k_r, v_r, kv_sems, acc_sc)
        if 0 < h < N_L + 1:
            origin_l = lax.rem(my + h, N_DEV)
            _attn_chunk(x16, comm_lq[h], comm_lo[h], origin_l,
                        k_ref, v_ref, k_l, v_l, kv_sems, acc_sc)

        for r in rdmas:
            r.wait_send()
        for r in rdmas:
            r.wait_recv()

    out_ref[:, :, :] = acc_sc[:, :].reshape(B_LOC, SQ, D_MODEL)


def kernel(x, Wq, K_ext, V_ext, Wo):
    my = lax.axis_index("i")
    k_loc = lax.dynamic_slice_in_dim(K_ext, my * B_LOC, B_LOC, axis=0)
    v_loc = lax.dynamic_slice_in_dim(V_ext, my * B_LOC, B_LOC, axis=0)
    k_t = jnp.transpose(k_loc, (0, 2, 1, 3))
    v_t = jnp.transpose(v_loc, (0, 2, 1, 3))

    bf = jnp.bfloat16
    return pl.pallas_call(
        _body,
        out_shape=jax.ShapeDtypeStruct((B_LOC, SQ, D_MODEL), jnp.float32),
        in_specs=[
            pl.BlockSpec(memory_space=pltpu.VMEM),
            pl.BlockSpec(memory_space=pltpu.VMEM),
            pl.BlockSpec(memory_space=pltpu.VMEM),
            pl.BlockSpec(memory_space=pl.ANY),
            pl.BlockSpec(memory_space=pl.ANY),
        ],
        out_specs=pl.BlockSpec(memory_space=pltpu.VMEM),
        scratch_shapes=[
            pltpu.VMEM((N_R + 1, D_MODEL, CH), bf),
            pltpu.VMEM((N_R + 1, CH, D_MODEL), bf),
            pltpu.VMEM((N_L + 1, D_MODEL, CH), bf),
            pltpu.VMEM((N_L + 1, CH, D_MODEL), bf),
            pltpu.VMEM((B_LOC * SQ, D_MODEL), jnp.float32),
            pltpu.VMEM((B_LOC, H_LOC, SKV, DH), jnp.float32),
            pltpu.VMEM((B_LOC, H_LOC, SKV, DH), jnp.float32),
            pltpu.VMEM((B_LOC, H_LOC, SKV, DH), jnp.float32),
            pltpu.VMEM((B_LOC, H_LOC, SKV, DH), jnp.float32),
            pltpu.SemaphoreType.DMA((N_R,)),
            pltpu.SemaphoreType.DMA((N_R,)),
            pltpu.SemaphoreType.DMA((N_R,)),
            pltpu.SemaphoreType.DMA((N_R,)),
            pltpu.SemaphoreType.DMA((N_L,)),
            pltpu.SemaphoreType.DMA((N_L,)),
            pltpu.SemaphoreType.DMA((N_L,)),
            pltpu.SemaphoreType.DMA((N_L,)),
            pltpu.SemaphoreType.DMA((2,)),
        ],
        compiler_params=pltpu.CompilerParams(
            collective_id=0,
            vmem_limit_bytes=100 * 1024 * 1024,
        ),
    )(x, Wq.astype(bf), Wo.astype(bf), k_t, v_t)


def _preflight():
    try:
        if len(jax.devices()) < N_DEV or jax.devices()[0].platform == "cpu":
            return
        import distributed_mesh_v7x as dm
        from jax.experimental.shard_map import shard_map
        from jax.sharding import NamedSharding, PartitionSpec as P

        mesh = dm.get_mesh("i", world_size=N_DEV)
        in_p = (P("i", None, None), P(None, "i"), P(None, None, None, None),
                P(None, None, None, None), P("i", None))
        shapes = ((64, 128, 512), (512, 8192), (64, 128, 128, 64),
                  (64, 128, 128, 64), (8192, 512))
        args = [jax.ShapeDtypeStruct(s, jnp.float32,
                                     sharding=NamedSharding(mesh, p))
                for s, p in zip(shapes, in_p)]
        wrapped = jax.jit(shard_map(
            kernel, mesh=mesh, in_specs=in_p,
            out_specs=P("i", None, None), check_rep=False))
        wrapped.lower(*args).compile()
    except Exception:
        pass
    try:
        for a in jax.live_arrays():
            a.block_until_ready()
    except Exception:
        pass


_preflight()


# device time: 173802 ns/iter; 2.6071x vs baseline; 1.4533x over previous
import jax
import jax.numpy as jnp
from jax import lax
from jax.experimental import pallas as pl
from jax.experimental.pallas import tpu as pltpu

N_DEV = 32
B_LOC = 2
SQ = 128
SKV = 128
H_LOC = 4
DH = 64
D_MODEL = 512
CH = H_LOC * DH

N_R = 16
N_L = 15

RING = (0, 8, 16, 24, 27, 19, 11, 3, 4, 12, 20, 28, 31, 23, 15, 7,
        6, 14, 22, 30, 29, 21, 13, 5, 2, 10, 18, 26, 25, 17, 9, 1)
POS = tuple(RING.index(i) for i in range(N_DEV))


def _attn_chunk(x16, q_wq, q_wo, origin, k_ref, v_ref, k_sc, v_sc,
                kv_sems, acc_sc):
    kdma = pltpu.make_async_copy(
        k_ref.at[:, pl.ds(origin * H_LOC, H_LOC)], k_sc, kv_sems.at[0])
    vdma = pltpu.make_async_copy(
        v_ref.at[:, pl.ds(origin * H_LOC, H_LOC)], v_sc, kv_sems.at[1])
    kdma.start()
    vdma.start()
    q_c = jnp.dot(x16, q_wq, preferred_element_type=jnp.float32)
    q3 = q_c.reshape(B_LOC, SQ, H_LOC, DH).transpose(0, 2, 1, 3).reshape(
        B_LOC * H_LOC, SQ, DH)
    kdma.wait()
    vdma.wait()
    k3 = k_sc[:, :, :, :].reshape(B_LOC * H_LOC, SKV, DH)
    v3 = v_sc[:, :, :, :].reshape(B_LOC * H_LOC, SKV, DH)
    s = lax.dot_general(
        q3, k3, (((2,), (2,)), ((0,), (0,))),
        preferred_element_type=jnp.float32,
    ) * 0.125
    m = jnp.max(s, axis=2, keepdims=True)
    w = jnp.exp(s - m)
    w = w / jnp.sum(w, axis=2, keepdims=True)
    ctx3 = lax.dot_general(
        w, v3, (((2,), (1,)), ((0,), (0,))),
        preferred_element_type=jnp.float32,
    )
    ctx = ctx3.reshape(B_LOC, H_LOC, SQ, DH).transpose(0, 2, 1, 3).reshape(
        B_LOC * SQ, CH)
    acc_sc[:, :] = acc_sc[:, :] + jnp.dot(
        ctx.astype(jnp.bfloat16), q_wo,
        preferred_element_type=jnp.float32)


def _body(nbr_ref, org_r_ref, org_l_ref, x_ref, wq_ref, wo_ref,
          k_ref, v_ref, out_ref,
          comm_rq, comm_ro, comm_lq, comm_lo, acc_sc,
          k_r, v_r, k_l, v_l,
          s_rq, r_rq, s_ro, r_ro, s_lq, r_lq, s_lo, r_lo, kv_sems):
    left = nbr_ref[0]
    right = nbr_ref[1]

    barrier_sem = pltpu.get_barrier_semaphore()
    for nbr in (left, right):
        pl.semaphore_signal(
            barrier_sem, inc=1,
            device_id=(nbr,), device_id_type=pl.DeviceIdType.MESH,
        )
    pl.semaphore_wait(barrier_sem, 2)

    comm_rq[0, :, :] = wq_ref[:, :]
    comm_ro[0, :, :] = wo_ref[:, :]
    comm_lq[0, :, :] = wq_ref[:, :]
    comm_lo[0, :, :] = wo_ref[:, :]
    acc_sc[:, :] = jnp.zeros((B_LOC * SQ, D_MODEL), jnp.float32)

    x16 = x_ref[:, :, :].reshape(B_LOC * SQ, D_MODEL).astype(jnp.bfloat16)

    for h in range(N_R + 1):
        rdmas = []
        if h < N_R:
            for (src, sem_s, sem_r) in ((comm_rq, s_rq, r_rq),
                                        (comm_ro, s_ro, r_ro)):
                r = pltpu.make_async_remote_copy(
                    src_ref=src.at[h], dst_ref=src.at[h + 1],
                    send_sem=sem_s.at[h], recv_sem=sem_r.at[h],
                    device_id=(right,), device_id_type=pl.DeviceIdType.MESH)
                r.start()
                rdmas.append(r)
        if h < N_L:
            for (src, sem_s, sem_r) in ((comm_lq, s_lq, r_lq),
                                        (comm_lo, s_lo, r_lo)):
                r = pltpu.make_async_remote_copy(
                    src_ref=src.at[h], dst_ref=src.at[h + 1],
                    send_sem=sem_s.at[h], recv_sem=sem_r.at[h],
                    device_id=(left,), device_id_type=pl.DeviceIdType.MESH)
                r.start()
                rdmas.append(r)

        _attn_chunk(x16, comm_rq[h], comm_ro[h], org_r_ref[h],
                    k_ref, v_ref, k_r, v_r, kv_sems, acc_sc)
        if 0 < h < N_L + 1:
            _attn_chunk(x16, comm_lq[h], comm_lo[h], org_l_ref[h],
                        k_ref, v_ref, k_l, v_l, kv_sems, acc_sc)

        for r in rdmas:
            r.wait_send()
        for r in rdmas:
            r.wait_recv()

    out_ref[:, :, :] = acc_sc[:, :].reshape(B_LOC, SQ, D_MODEL)


def kernel(x, Wq, K_ext, V_ext, Wo):
    my = lax.axis_index("i")
    k_loc = lax.dynamic_slice_in_dim(K_ext, my * B_LOC, B_LOC, axis=0)
    v_loc = lax.dynamic_slice_in_dim(V_ext, my * B_LOC, B_LOC, axis=0)
    k_t = jnp.transpose(k_loc, (0, 2, 1, 3))
    v_t = jnp.transpose(v_loc, (0, 2, 1, 3))

    ring = jnp.asarray(RING, jnp.int32)
    p = jnp.asarray(POS, jnp.int32)[my]
    nbrs = jnp.stack([ring[(p + N_DEV - 1) % N_DEV],
                      ring[(p + 1) % N_DEV]])
    org_r = ring[(p + N_DEV - jnp.arange(N_R + 1, dtype=jnp.int32)) % N_DEV]
    org_l = ring[(p + jnp.arange(N_L + 1, dtype=jnp.int32)) % N_DEV]

    bf = jnp.bfloat16
    return pl.pallas_call(
        _body,
        out_shape=jax.ShapeDtypeStruct((B_LOC, SQ, D_MODEL), jnp.float32),
        in_specs=[
            pl.BlockSpec(memory_space=pltpu.SMEM),
            pl.BlockSpec(memory_space=pltpu.SMEM),
            pl.BlockSpec(memory_space=pltpu.SMEM),
            pl.BlockSpec(memory_space=pltpu.VMEM),
            pl.BlockSpec(memory_space=pltpu.VMEM),
            pl.BlockSpec(memory_space=pltpu.VMEM),
            pl.BlockSpec(memory_space=pl.ANY),
            pl.BlockSpec(memory_space=pl.ANY),
        ],
        out_specs=pl.BlockSpec(memory_space=pltpu.VMEM),
        scratch_shapes=[
            pltpu.VMEM((N_R + 1, D_MODEL, CH), bf),
            pltpu.VMEM((N_R + 1, CH, D_MODEL), bf),
            pltpu.VMEM((N_L + 1, D_MODEL, CH), bf),
            pltpu.VMEM((N_L + 1, CH, D_MODEL), bf),
            pltpu.VMEM((B_LOC * SQ, D_MODEL), jnp.float32),
            pltpu.VMEM((B_LOC, H_LOC, SKV, DH), jnp.float32),
            pltpu.VMEM((B_LOC, H_LOC, SKV, DH), jnp.float32),
            pltpu.VMEM((B_LOC, H_LOC, SKV, DH), jnp.float32),
            pltpu.VMEM((B_LOC, H_LOC, SKV, DH), jnp.float32),
            pltpu.SemaphoreType.DMA((N_R,)),
            pltpu.SemaphoreType.DMA((N_R,)),
            pltpu.SemaphoreType.DMA((N_R,)),
            pltpu.SemaphoreType.DMA((N_R,)),
            pltpu.SemaphoreType.DMA((N_L,)),
            pltpu.SemaphoreType.DMA((N_L,)),
            pltpu.SemaphoreType.DMA((N_L,)),
            pltpu.SemaphoreType.DMA((N_L,)),
            pltpu.SemaphoreType.DMA((2,)),
        ],
        compiler_params=pltpu.CompilerParams(
            collective_id=0,
            vmem_limit_bytes=100 * 1024 * 1024,
        ),
    )(nbrs, org_r, org_l, x, Wq.astype(bf), Wo.astype(bf), k_t, v_t)


def _preflight():
    try:
        if len(jax.devices()) < N_DEV or jax.devices()[0].platform == "cpu":
            return
        import distributed_mesh_v7x as dm
        from jax.experimental.shard_map import shard_map
        from jax.sharding import NamedSharding, PartitionSpec as P

        mesh = dm.get_mesh("i", world_size=N_DEV)
        in_p = (P("i", None, None), P(None, "i"), P(None, None, None, None),
                P(None, None, None, None), P("i", None))
        shapes = ((64, 128, 512), (512, 8192), (64, 128, 128, 64),
                  (64, 128, 128, 64), (8192, 512))
        args = [jax.ShapeDtypeStruct(s, jnp.float32,
                                     sharding=NamedSharding(mesh, p))
                for s, p in zip(shapes, in_p)]
        wrapped = jax.jit(shard_map(
            kernel, mesh=mesh, in_specs=in_p,
            out_specs=P("i", None, None), check_rep=False))
        wrapped.lower(*args).compile()
    except Exception:
        pass
    try:
        for a in jax.live_arrays():
            a.block_until_ready()
    except Exception:
        pass


_preflight()


# device time: 168768 ns/iter; 2.6849x vs baseline; 1.0298x over previous
import jax
import jax.numpy as jnp
from jax import lax
from jax.experimental import pallas as pl
from jax.experimental.pallas import tpu as pltpu

N_DEV = 32
B_LOC = 2
SQ = 128
SKV = 128
H_LOC = 4
DH = 64
D_MODEL = 512
CH = H_LOC * DH

N_R = 16
N_L = 15

RING = (0, 8, 16, 24, 27, 19, 11, 3, 4, 12, 20, 28, 31, 23, 15, 7,
        6, 14, 22, 30, 29, 21, 13, 5, 2, 10, 18, 26, 25, 17, 9, 1)
POS = tuple(RING.index(i) for i in range(N_DEV))


def _attn_chunk(x16, w_c, origin, k_ref, v_ref, k_sc, v_sc,
                kv_sems, acc_sc):
    kdma = pltpu.make_async_copy(
        k_ref.at[:, pl.ds(origin * H_LOC, H_LOC)], k_sc, kv_sems.at[0])
    vdma = pltpu.make_async_copy(
        v_ref.at[:, pl.ds(origin * H_LOC, H_LOC)], v_sc, kv_sems.at[1])
    kdma.start()
    vdma.start()
    q_c = jnp.dot(x16, w_c[:D_MODEL], preferred_element_type=jnp.float32)
    q3 = q_c.reshape(B_LOC, SQ, H_LOC, DH).transpose(0, 2, 1, 3).reshape(
        B_LOC * H_LOC, SQ, DH).astype(jnp.bfloat16)
    kdma.wait()
    vdma.wait()
    k3 = k_sc[:, :, :, :].reshape(B_LOC * H_LOC, SKV, DH)
    v3 = v_sc[:, :, :, :].reshape(B_LOC * H_LOC, SKV, DH)
    s = lax.dot_general(
        q3, k3, (((2,), (2,)), ((0,), (0,))),
        preferred_element_type=jnp.float32,
    ) * 0.125
    m = jnp.max(s, axis=2, keepdims=True)
    w = jnp.exp(s - m)
    w = (w / jnp.sum(w, axis=2, keepdims=True)).astype(jnp.bfloat16)
    ctx3 = lax.dot_general(
        w, v3, (((2,), (1,)), ((0,), (0,))),
        preferred_element_type=jnp.float32,
    )
    ctx = ctx3.reshape(B_LOC, H_LOC, SQ, DH).transpose(0, 2, 1, 3).reshape(
        B_LOC * SQ, CH).astype(jnp.bfloat16)
    acc_sc[:, :] = acc_sc[:, :] + lax.dot_general(
        ctx, w_c[D_MODEL:], (((1,), (1,)), ((), ())),
        preferred_element_type=jnp.float32)


def _body(nbr_ref, org_r_ref, org_l_ref, x_ref, w_ref,
          k_ref, v_ref, out_ref,
          comm_r, comm_l, acc_sc,
          k_r, v_r, k_l, v_l,
          s_r, r_r, s_l, r_l, kv_sems):
    left = nbr_ref[0]
    right = nbr_ref[1]

    barrier_sem = pltpu.get_barrier_semaphore()
    for nbr in (left, right):
        pl.semaphore_signal(
            barrier_sem, inc=1,
            device_id=(nbr,), device_id_type=pl.DeviceIdType.MESH,
        )
    pl.semaphore_wait(barrier_sem, 2)

    comm_r[0, :, :] = w_ref[:, :]
    comm_l[0, :, :] = w_ref[:, :]
    acc_sc[:, :] = jnp.zeros((B_LOC * SQ, D_MODEL), jnp.float32)

    x16 = x_ref[:, :, :].reshape(B_LOC * SQ, D_MODEL).astype(jnp.bfloat16)

    for h in range(N_R + 1):
        rdmas = []
        if h < N_R:
            r = pltpu.make_async_remote_copy(
                src_ref=comm_r.at[h], dst_ref=comm_r.at[h + 1],
                send_sem=s_r.at[h], recv_sem=r_r.at[h],
                device_id=(right,), device_id_type=pl.DeviceIdType.MESH)
            r.start()
            rdmas.append(r)
        if h < N_L:
            r = pltpu.make_async_remote_copy(
                src_ref=comm_l.at[h], dst_ref=comm_l.at[h + 1],
                send_sem=s_l.at[h], recv_sem=r_l.at[h],
                device_id=(left,), device_id_type=pl.DeviceIdType.MESH)
            r.start()
            rdmas.append(r)

        _attn_chunk(x16, comm_r[h], org_r_ref[h],
                    k_ref, v_ref, k_r, v_r, kv_sems, acc_sc)
        if 0 < h < N_L + 1:
            _attn_chunk(x16, comm_l[h], org_l_ref[h],
                        k_ref, v_ref, k_l, v_l, kv_sems, acc_sc)

        for r in rdmas:
            r.wait_send()
        for r in rdmas:
            r.wait_recv()

    out_ref[:, :, :] = acc_sc[:, :].reshape(B_LOC, SQ, D_MODEL)


def kernel(x, Wq, K_ext, V_ext, Wo):
    my = lax.axis_index("i")
    k_loc = lax.dynamic_slice_in_dim(K_ext, my * B_LOC, B_LOC, axis=0)
    v_loc = lax.dynamic_slice_in_dim(V_ext, my * B_LOC, B_LOC, axis=0)
    k_t = jnp.transpose(k_loc, (0, 2, 1, 3)).astype(jnp.bfloat16)
    v_t = jnp.transpose(v_loc, (0, 2, 1, 3)).astype(jnp.bfloat16)

    ring = jnp.asarray(RING, jnp.int32)
    p = jnp.asarray(POS, jnp.int32)[my]
    nbrs = jnp.stack([ring[(p + N_DEV - 1) % N_DEV],
                      ring[(p + 1) % N_DEV]])
    org_r = ring[(p + N_DEV - jnp.arange(N_R + 1, dtype=jnp.int32)) % N_DEV]
    org_l = ring[(p + jnp.arange(N_L + 1, dtype=jnp.int32)) % N_DEV]

    bf = jnp.bfloat16
    return pl.pallas_call(
        _body,
        out_shape=jax.ShapeDtypeStruct((B_LOC, SQ, D_MODEL), jnp.float32),
        in_specs=[
            pl.BlockSpec(memory_space=pltpu.SMEM),
            pl.BlockSpec(memory_space=pltpu.SMEM),
            pl.BlockSpec(memory_space=pltpu.SMEM),
            pl.BlockSpec(memory_space=pltpu.VMEM),
            pl.BlockSpec(memory_space=pltpu.VMEM),
            pl.BlockSpec(memory_space=pl.ANY),
            pl.BlockSpec(memory_space=pl.ANY),
        ],
        out_specs=pl.BlockSpec(memory_space=pltpu.VMEM),
        scratch_shapes=[
            pltpu.VMEM((N_R + 1, 2 * D_MODEL, CH), bf),
            pltpu.VMEM((N_L + 1, 2 * D_MODEL, CH), bf),
            pltpu.VMEM((B_LOC * SQ, D_MODEL), jnp.float32),
            pltpu.VMEM((B_LOC, H_LOC, SKV, DH), bf),
            pltpu.VMEM((B_LOC, H_LOC, SKV, DH), bf),
            pltpu.VMEM((B_LOC, H_LOC, SKV, DH), bf),
            pltpu.VMEM((B_LOC, H_LOC, SKV, DH), bf),
            pltpu.SemaphoreType.DMA((N_R,)),
            pltpu.SemaphoreType.DMA((N_R,)),
            pltpu.SemaphoreType.DMA((N_L,)),
            pltpu.SemaphoreType.DMA((N_L,)),
            pltpu.SemaphoreType.DMA((2,)),
        ],
        compiler_params=pltpu.CompilerParams(
            collective_id=0,
            vmem_limit_bytes=100 * 1024 * 1024,
        ),
    )(nbrs, org_r, org_l, x,
      jnp.concatenate([Wq.astype(bf), Wo.T.astype(bf)], axis=0), k_t, v_t)


def _preflight():
    try:
        if len(jax.devices()) < N_DEV or jax.devices()[0].platform == "cpu":
            return
        import distributed_mesh_v7x as dm
        from jax.experimental.shard_map import shard_map
        from jax.sharding import NamedSharding, PartitionSpec as P

        mesh = dm.get_mesh("i", world_size=N_DEV)
        in_p = (P("i", None, None), P(None, "i"), P(None, None, None, None),
                P(None, None, None, None), P("i", None))
        shapes = ((64, 128, 512), (512, 8192), (64, 128, 128, 64),
                  (64, 128, 128, 64), (8192, 512))
        args = [jax.ShapeDtypeStruct(s, jnp.float32,
                                     sharding=NamedSharding(mesh, p))
                for s, p in zip(shapes, in_p)]
        wrapped = jax.jit(shard_map(
            kernel, mesh=mesh, in_specs=in_p,
            out_specs=P("i", None, None), check_rep=False))
        wrapped.lower(*args).compile()
    except Exception:
        pass
    try:
        for a in jax.live_arrays():
            a.block_until_ready()
    except Exception:
        pass


_preflight()


# device time: 146973 ns/iter; 3.0830x vs baseline; 1.1483x over previous
import jax
import jax.numpy as jnp
from jax import lax
from jax.experimental import pallas as pl
from jax.experimental.pallas import tpu as pltpu

N_DEV = 32
B_LOC = 2
SQ = 128
SKV = 128
H_LOC = 4
DH = 64
D_MODEL = 512
CH = H_LOC * DH

N_R = 16
N_L = 15

RING = (0, 8, 16, 24, 27, 19, 11, 3, 4, 12, 20, 28, 31, 23, 15, 7,
        6, 14, 22, 30, 29, 21, 13, 5, 2, 10, 18, 26, 25, 17, 9, 1)
POS = tuple(RING.index(i) for i in range(N_DEV))


def _attn_ctx(x16, wq_c, origin, k_ref, v_ref, k_sc, v_sc, kv_sems):
    kdma = pltpu.make_async_copy(
        k_ref.at[:, pl.ds(origin * H_LOC, H_LOC)], k_sc, kv_sems.at[0])
    vdma = pltpu.make_async_copy(
        v_ref.at[:, pl.ds(origin * H_LOC, H_LOC)], v_sc, kv_sems.at[1])
    kdma.start()
    vdma.start()
    q_c = jnp.dot(x16, wq_c, preferred_element_type=jnp.float32)
    q3 = q_c.reshape(B_LOC, SQ, H_LOC, DH).transpose(0, 2, 1, 3).reshape(
        B_LOC * H_LOC, SQ, DH).astype(jnp.bfloat16)
    kdma.wait()
    vdma.wait()
    k3 = k_sc[:, :, :, :].reshape(B_LOC * H_LOC, SKV, DH)
    v3 = v_sc[:, :, :, :].reshape(B_LOC * H_LOC, SKV, DH)
    s = lax.dot_general(
        q3, k3, (((2,), (2,)), ((0,), (0,))),
        preferred_element_type=jnp.float32,
    ) * 0.125
    m = jnp.max(s, axis=2, keepdims=True)
    w = jnp.exp(s - m)
    w = (w / jnp.sum(w, axis=2, keepdims=True)).astype(jnp.bfloat16)
    ctx3 = lax.dot_general(
        w, v3, (((2,), (1,)), ((0,), (0,))),
        preferred_element_type=jnp.float32,
    )
    return ctx3.reshape(B_LOC, H_LOC, SQ, DH).transpose(0, 2, 1, 3).reshape(
        B_LOC * SQ, CH).astype(jnp.bfloat16)


def _body(nbr_ref, org_r_ref, org_l_ref, x_ref, wq_ref, wo_ref,
          k_ref, v_ref, out_ref,
          cq_r, co_r, cq_l, co_l, acc_sc,
          k_r, v_r, k_l, v_l,
          sq_r, rq_r, so_r, ro_r, sq_l, rq_l, so_l, ro_l, kv_sems):
    left = nbr_ref[0]
    right = nbr_ref[1]

    barrier_sem = pltpu.get_barrier_semaphore()
    for nbr in (left, right):
        pl.semaphore_signal(
            barrier_sem, inc=1,
            device_id=(nbr,), device_id_type=pl.DeviceIdType.MESH,
        )
    pl.semaphore_wait(barrier_sem, 2)

    cq_r[0, :, :] = wq_ref[:, :]
    co_r[0, :, :] = wo_ref[:, :]
    cq_l[0, :, :] = wq_ref[:, :]
    co_l[0, :, :] = wo_ref[:, :]
    acc_sc[:, :] = jnp.zeros((B_LOC * SQ, D_MODEL), jnp.float32)

    x16 = x_ref[:, :, :].reshape(B_LOC * SQ, D_MODEL).astype(jnp.bfloat16)

    def mk(buf, sems_s, sems_r, h, dev):
        return pltpu.make_async_remote_copy(
            src_ref=buf.at[h], dst_ref=buf.at[h + 1],
            send_sem=sems_s.at[h], recv_sem=sems_r.at[h],
            device_id=(dev,), device_id_type=pl.DeviceIdType.MESH)

    wqr, wql, wor, wol = [], [], [], []
    for h in range(N_R + 1):
        if h >= 1 and h - 1 < N_R:
            wqr[h - 1].wait_recv()
        if h >= 1 and h - 1 < N_L:
            wql[h - 1].wait_recv()
        if h < N_R:
            d = mk(cq_r, sq_r, rq_r, h, right)
            d.start()
            wqr.append(d)
        if h < N_L:
            d = mk(cq_l, sq_l, rq_l, h, left)
            d.start()
            wql.append(d)

        ctx_r = _attn_ctx(x16, cq_r[h], org_r_ref[h],
                          k_ref, v_ref, k_r, v_r, kv_sems)
        ctx_l = None
        if 0 < h < N_L + 1:
            ctx_l = _attn_ctx(x16, cq_l[h], org_l_ref[h],
                              k_ref, v_ref, k_l, v_l, kv_sems)

        if h >= 1 and h - 1 < N_R:
            wor[h - 1].wait_recv()
        if h >= 1 and h - 1 < N_L:
            wol[h - 1].wait_recv()
        if h < N_R:
            d = mk(co_r, so_r, ro_r, h, right)
            d.start()
            wor.append(d)
        if h < N_L:
            d = mk(co_l, so_l, ro_l, h, left)
            d.start()
            wol.append(d)

        acc = acc_sc[:, :] + lax.dot_general(
            ctx_r, co_r[h], (((1,), (1,)), ((), ())),
            preferred_element_type=jnp.float32)
        if ctx_l is not None:
            acc = acc + lax.dot_general(
                ctx_l, co_l[h], (((1,), (1,)), ((), ())),
                preferred_element_type=jnp.float32)
        acc_sc[:, :] = acc

    for d in wqr + wql + wor + wol:
        d.wait_send()

    out_ref[:, :, :] = acc_sc[:, :].reshape(B_LOC, SQ, D_MODEL)


def kernel(x, Wq, K_ext, V_ext, Wo):
    my = lax.axis_index("i")
    k_loc = lax.dynamic_slice_in_dim(K_ext, my * B_LOC, B_LOC, axis=0)
    v_loc = lax.dynamic_slice_in_dim(V_ext, my * B_LOC, B_LOC, axis=0)
    k_t = jnp.transpose(k_loc, (0, 2, 1, 3)).astype(jnp.bfloat16)
    v_t = jnp.transpose(v_loc, (0, 2, 1, 3)).astype(jnp.bfloat16)

    ring = jnp.asarray(RING, jnp.int32)
    p = jnp.asarray(POS, jnp.int32)[my]
    nbrs = jnp.stack([ring[(p + N_DEV - 1) % N_DEV],
                      ring[(p + 1) % N_DEV]])
    org_r = ring[(p + N_DEV - jnp.arange(N_R + 1, dtype=jnp.int32)) % N_DEV]
    org_l = ring[(p + jnp.arange(N_L + 1, dtype=jnp.int32)) % N_DEV]

    bf = jnp.bfloat16
    return pl.pallas_call(
        _body,
        out_shape=jax.ShapeDtypeStruct((B_LOC, SQ, D_MODEL), jnp.float32),
        in_specs=[
            pl.BlockSpec(memory_space=pltpu.SMEM),
            pl.BlockSpec(memory_space=pltpu.SMEM),
            pl.BlockSpec(memory_space=pltpu.SMEM),
            pl.BlockSpec(memory_space=pltpu.VMEM),
            pl.BlockSpec(memory_space=pltpu.VMEM),
            pl.BlockSpec(memory_space=pltpu.VMEM),
            pl.BlockSpec(memory_space=pl.ANY),
            pl.BlockSpec(memory_space=pl.ANY),
        ],
        out_specs=pl.BlockSpec(memory_space=pltpu.VMEM),
        scratch_shapes=[
            pltpu.VMEM((N_R + 1, D_MODEL, CH), bf),
            pltpu.VMEM((N_R + 1, D_MODEL, CH), bf),
            pltpu.VMEM((N_L + 1, D_MODEL, CH), bf),
            pltpu.VMEM((N_L + 1, D_MODEL, CH), bf),
            pltpu.VMEM((B_LOC * SQ, D_MODEL), jnp.float32),
            pltpu.VMEM((B_LOC, H_LOC, SKV, DH), bf),
            pltpu.VMEM((B_LOC, H_LOC, SKV, DH), bf),
            pltpu.VMEM((B_LOC, H_LOC, SKV, DH), bf),
            pltpu.VMEM((B_LOC, H_LOC, SKV, DH), bf),
            pltpu.SemaphoreType.DMA((N_R,)),
            pltpu.SemaphoreType.DMA((N_R,)),
            pltpu.SemaphoreType.DMA((N_R,)),
            pltpu.SemaphoreType.DMA((N_R,)),
            pltpu.SemaphoreType.DMA((N_L,)),
            pltpu.SemaphoreType.DMA((N_L,)),
            pltpu.SemaphoreType.DMA((N_L,)),
            pltpu.SemaphoreType.DMA((N_L,)),
            pltpu.SemaphoreType.DMA((2,)),
        ],
        compiler_params=pltpu.CompilerParams(
            collective_id=0,
            vmem_limit_bytes=100 * 1024 * 1024,
        ),
    )(nbrs, org_r, org_l, x, Wq.astype(bf), Wo.T.astype(bf), k_t, v_t)


def _preflight():
    try:
        if len(jax.devices()) < N_DEV or jax.devices()[0].platform == "cpu":
            return
        import distributed_mesh_v7x as dm
        from jax.experimental.shard_map import shard_map
        from jax.sharding import NamedSharding, PartitionSpec as P

        mesh = dm.get_mesh("i", world_size=N_DEV)
        in_p = (P("i", None, None), P(None, "i"), P(None, None, None, None),
                P(None, None, None, None), P("i", None))
        shapes = ((64, 128, 512), (512, 8192), (64, 128, 128, 64),
                  (64, 128, 128, 64), (8192, 512))
        args = [jax.ShapeDtypeStruct(s, jnp.float32,
                                     sharding=NamedSharding(mesh, p))
                for s, p in zip(shapes, in_p)]
        wrapped = jax.jit(shard_map(
            kernel, mesh=mesh, in_specs=in_p,
            out_specs=P("i", None, None), check_rep=False))
        wrapped.lower(*args).compile()
    except Exception:
        pass
    try:
        for a in jax.live_arrays():
            a.block_until_ready()
    except Exception:
        pass


_preflight()


# device time: 139937 ns/iter; 3.2380x vs baseline; 1.0503x over previous
import jax
import jax.numpy as jnp
from jax import lax
from jax.experimental import pallas as pl
from jax.experimental.pallas import tpu as pltpu

N_DEV = 32
B_LOC = 2
SQ = 128
SKV = 128
H_LOC = 4
DH = 64
D_MODEL = 512
CH = H_LOC * DH

N_R = 16
N_L = 15

RING = (0, 8, 16, 24, 27, 19, 11, 3, 4, 12, 20, 28, 31, 23, 15, 7,
        6, 14, 22, 30, 29, 21, 13, 5, 2, 10, 18, 26, 25, 17, 9, 1)
POS = tuple(RING.index(i) for i in range(N_DEV))


def _kv_dmas(origin, k_ref, v_ref, k_sc, v_sc, kv_sems, base):
    kdma = pltpu.make_async_copy(
        k_ref.at[:, pl.ds(origin * H_LOC, H_LOC)], k_sc, kv_sems.at[base])
    vdma = pltpu.make_async_copy(
        v_ref.at[:, pl.ds(origin * H_LOC, H_LOC)], v_sc, kv_sems.at[base + 1])
    kdma.start()
    vdma.start()
    return kdma, vdma


def _attn_ctx(x16, wq_c, kv, k_sc, v_sc):
    q_c = jnp.dot(x16, wq_c, preferred_element_type=jnp.float32)
    q3 = q_c.reshape(B_LOC, SQ, H_LOC, DH).transpose(0, 2, 1, 3).reshape(
        B_LOC * H_LOC, SQ, DH).astype(jnp.bfloat16)
    for d in kv:
        d.wait()
    k3 = k_sc[:, :, :, :].reshape(B_LOC * H_LOC, SKV, DH)
    v3 = v_sc[:, :, :, :].reshape(B_LOC * H_LOC, SKV, DH)
    s = lax.dot_general(
        q3, k3, (((2,), (2,)), ((0,), (0,))),
        preferred_element_type=jnp.float32,
    ) * 0.125
    m = jnp.max(s, axis=2, keepdims=True)
    w = jnp.exp(s - m)
    w = (w / jnp.sum(w, axis=2, keepdims=True)).astype(jnp.bfloat16)
    ctx3 = lax.dot_general(
        w, v3, (((2,), (1,)), ((0,), (0,))),
        preferred_element_type=jnp.float32,
    )
    return ctx3.reshape(B_LOC, H_LOC, SQ, DH).transpose(0, 2, 1, 3).reshape(
        B_LOC * SQ, CH).astype(jnp.bfloat16)


def _body(nbr_ref, org_r_ref, org_l_ref, x_ref, wq_ref, wo_ref,
          k_ref, v_ref, out_ref,
          cq_r, co_r, cq_l, co_l, acc_sc,
          k_r, v_r, k_l, v_l,
          sq_r, rq_r, so_r, ro_r, sq_l, rq_l, so_l, ro_l, kv_sems):
    left = nbr_ref[0]
    right = nbr_ref[1]

    barrier_sem = pltpu.get_barrier_semaphore()
    for nbr in (left, right):
        pl.semaphore_signal(
            barrier_sem, inc=1,
            device_id=(nbr,), device_id_type=pl.DeviceIdType.MESH,
        )
    pl.semaphore_wait(barrier_sem, 2)

    cq_r[0, :, :] = wq_ref[:, :]
    co_r[0, :, :] = wo_ref[:, :]
    cq_l[0, :, :] = wq_ref[:, :]
    co_l[0, :, :] = wo_ref[:, :]
    acc_sc[:, :] = jnp.zeros((B_LOC * SQ, D_MODEL), jnp.float32)

    x16 = x_ref[:, :, :].reshape(B_LOC * SQ, D_MODEL).astype(jnp.bfloat16)

    def mk(buf, sems_s, sems_r, h, dev):
        return pltpu.make_async_remote_copy(
            src_ref=buf.at[h], dst_ref=buf.at[h + 1],
            send_sem=sems_s.at[h], recv_sem=sems_r.at[h],
            device_id=(dev,), device_id_type=pl.DeviceIdType.MESH)

    wqr, wql, wor, wol = [], [], [], []
    for h in range(N_R + 1):
        kv_r = _kv_dmas(org_r_ref[h], k_ref, v_ref, k_r, v_r, kv_sems, 0)
        kv_l = None
        if 0 < h < N_L + 1:
            kv_l = _kv_dmas(org_l_ref[h], k_ref, v_ref, k_l, v_l, kv_sems, 2)

        if h >= 1 and h - 1 < N_R:
            wqr[h - 1].wait_recv()
        if h >= 1 and h - 1 < N_L:
            wql[h - 1].wait_recv()
        if h < N_R:
            d = mk(cq_r, sq_r, rq_r, h, right)
            d.start()
            wqr.append(d)
        if h < N_L:
            d = mk(cq_l, sq_l, rq_l, h, left)
            d.start()
            wql.append(d)

        ctx_r = _attn_ctx(x16, cq_r[h], kv_r, k_r, v_r)
        ctx_l = None
        if kv_l is not None:
            ctx_l = _attn_ctx(x16, cq_l[h], kv_l, k_l, v_l)

        if h >= 1 and h - 1 < N_R:
            wor[h - 1].wait_recv()
        if h >= 1 and h - 1 < N_L:
            wol[h - 1].wait_recv()
        if h < N_R:
            d = mk(co_r, so_r, ro_r, h, right)
            d.start()
            wor.append(d)
        if h < N_L:
            d = mk(co_l, so_l, ro_l, h, left)
            d.start()
            wol.append(d)

        acc = acc_sc[:, :] + lax.dot_general(
            ctx_r, co_r[h], (((1,), (1,)), ((), ())),
            preferred_element_type=jnp.float32)
        if ctx_l is not None:
            acc = acc + lax.dot_general(
                ctx_l, co_l[h], (((1,), (1,)), ((), ())),
                preferred_element_type=jnp.float32)
        acc_sc[:, :] = acc

    for d in wqr + wql + wor + wol:
        d.wait_send()

    out_ref[:, :, :] = acc_sc[:, :].reshape(B_LOC, SQ, D_MODEL)


def kernel(x, Wq, K_ext, V_ext, Wo):
    my = lax.axis_index("i")
    k_loc = lax.dynamic_slice_in_dim(K_ext, my * B_LOC, B_LOC, axis=0)
    v_loc = lax.dynamic_slice_in_dim(V_ext, my * B_LOC, B_LOC, axis=0)
    k_t = jnp.transpose(k_loc, (0, 2, 1, 3)).astype(jnp.bfloat16)
    v_t = jnp.transpose(v_loc, (0, 2, 1, 3)).astype(jnp.bfloat16)

    ring = jnp.asarray(RING, jnp.int32)
    p = jnp.asarray(POS, jnp.int32)[my]
    nbrs = jnp.stack([ring[(p + N_DEV - 1) % N_DEV],
                      ring[(p + 1) % N_DEV]])
    org_r = ring[(p + N_DEV - jnp.arange(N_R + 1, dtype=jnp.int32)) % N_DEV]
    org_l = ring[(p + jnp.arange(N_L + 1, dtype=jnp.int32)) % N_DEV]

    bf = jnp.bfloat16
    return pl.pallas_call(
        _body,
        out_shape=jax.ShapeDtypeStruct((B_LOC, SQ, D_MODEL), jnp.float32),
        in_specs=[
            pl.BlockSpec(memory_space=pltpu.SMEM),
            pl.BlockSpec(memory_space=pltpu.SMEM),
            pl.BlockSpec(memory_space=pltpu.SMEM),
            pl.BlockSpec(memory_space=pltpu.VMEM),
            pl.BlockSpec(memory_space=pltpu.VMEM),
            pl.BlockSpec(memory_space=pltpu.VMEM),
            pl.BlockSpec(memory_space=pl.ANY),
            pl.BlockSpec(memory_space=pl.ANY),
        ],
        out_specs=pl.BlockSpec(memory_space=pltpu.VMEM),
        scratch_shapes=[
            pltpu.VMEM((N_R + 1, D_MODEL, CH), bf),
            pltpu.VMEM((N_R + 1, D_MODEL, CH), bf),
            pltpu.VMEM((N_L + 1, D_MODEL, CH), bf),
            pltpu.VMEM((N_L + 1, D_MODEL, CH), bf),
            pltpu.VMEM((B_LOC * SQ, D_MODEL), jnp.float32),
            pltpu.VMEM((B_LOC, H_LOC, SKV, DH), bf),
            pltpu.VMEM((B_LOC, H_LOC, SKV, DH), bf),
            pltpu.VMEM((B_LOC, H_LOC, SKV, DH), bf),
            pltpu.VMEM((B_LOC, H_LOC, SKV, DH), bf),
            pltpu.SemaphoreType.DMA((N_R,)),
            pltpu.SemaphoreType.DMA((N_R,)),
            pltpu.SemaphoreType.DMA((N_R,)),
            pltpu.SemaphoreType.DMA((N_R,)),
            pltpu.SemaphoreType.DMA((N_L,)),
            pltpu.SemaphoreType.DMA((N_L,)),
            pltpu.SemaphoreType.DMA((N_L,)),
            pltpu.SemaphoreType.DMA((N_L,)),
            pltpu.SemaphoreType.DMA((4,)),
        ],
        compiler_params=pltpu.CompilerParams(
            collective_id=0,
            vmem_limit_bytes=100 * 1024 * 1024,
        ),
    )(nbrs, org_r, org_l, x, Wq.astype(bf), Wo.T.astype(bf), k_t, v_t)


def _preflight():
    try:
        if len(jax.devices()) < N_DEV or jax.devices()[0].platform == "cpu":
            return
        import distributed_mesh_v7x as dm
        from jax.experimental.shard_map import shard_map
        from jax.sharding import NamedSharding, PartitionSpec as P

        mesh = dm.get_mesh("i", world_size=N_DEV)
        in_p = (P("i", None, None), P(None, "i"), P(None, None, None, None),
                P(None, None, None, None), P("i", None))
        shapes = ((64, 128, 512), (512, 8192), (64, 128, 128, 64),
                  (64, 128, 128, 64), (8192, 512))
        args = [jax.ShapeDtypeStruct(s, jnp.float32,
                                     sharding=NamedSharding(mesh, p))
                for s, p in zip(shapes, in_p)]
        wrapped = jax.jit(shard_map(
            kernel, mesh=mesh, in_specs=in_p,
            out_specs=P("i", None, None), check_rep=False))
        wrapped.lower(*args).compile()
    except Exception:
        pass
    try:
        for a in jax.live_arrays():
            a.block_until_ready()
    except Exception:
        pass


_preflight()
